# baseline (speedup 1.0000x reference)
import functools

import jax
import jax.numpy as jnp
from jax import lax
from jax.experimental import pallas as pl
from jax.experimental.pallas import tpu as pltpu
from jax.experimental.pallas import tpu_sc as plsc

L = 16
NC = 2
NS = 16
SLOTS = 8


def _sc_mesh():
  return plsc.VectorSubcoreMesh(
      core_axis_name="c", subcore_axis_name="s", num_cores=NC, num_subcores=NS)


def _stage1_scores_copy(w_ref, x_ref, o_ref, s_ref):
  xb = x_ref[...]
  o_ref[...] = xb
  s_ref[...] = jnp.dot(xb, w_ref[...], preferred_element_type=jnp.float32)


def _stage4_layer(sel_ref, w_ref, b_ref, v_ref, o_ref):
  sb = sel_ref[...]
  p = jnp.dot(sb, w_ref[...], preferred_element_type=jnp.float32) + b_ref[...]
  sg = jax.nn.sigmoid(v_ref[...])
  o_ref[...] = sb + (p * sg - sb * sg)


def _make_topk(B, T, D, K, KP):
  CH = T // SLOTS
  NV = CH // L
  NVT = T // L
  WPT = K // SLOTS

  @functools.partial(
      pl.kernel,
      out_type=[
          jax.ShapeDtypeStruct((B * KP,), jnp.int32),
          jax.ShapeDtypeStruct((B * KP,), jnp.float32),
          jax.ShapeDtypeStruct((B * K, D), jnp.float32),
      ],
      mesh=_sc_mesh(),
      compiler_params=pltpu.CompilerParams(needs_layout_passes=False),
      scratch_types=[
          pltpu.VMEM((T,), jnp.float32),
          pltpu.VMEM((T,), jnp.uint32),
          pltpu.VMEM((T + L,), jnp.uint32),
          pltpu.VMEM((256,), jnp.int32),
          pltpu.VMEM((WPT + L,), jnp.int32),
          pltpu.VMEM((WPT + L,), jnp.float32),
          pltpu.VMEM((WPT, D), jnp.float32),
          pltpu.SemaphoreType.DMA,
      ],
  )
  def _topk(scores_hbm, x_hbm, rid_hbm, val_hbm, sel_hbm, svals, ukeys, cand,
            hist, myrid, myval, rows_v, sem):
    c = lax.axis_index("c")
    s = lax.axis_index("s")
    bl = s // SLOTS
    slot = s % SLOTS
    batch = c * 2 + bl
    base_flat = batch * T + slot * CH

    with jax.named_scope("tk_load"):
      pltpu.sync_copy(scores_hbm.at[pl.ds(batch * T, T)], svals)

    iota = lax.iota(jnp.int32, L)
    z16 = jnp.zeros((L,), jnp.int32)
    ones16 = jnp.ones((L,), jnp.int32)
    kk = jnp.int32(K)

    def _init(j, _):
      v = svals[pl.ds(j * L, L)]
      bts = plsc.bitcast(v, jnp.uint32)
      neg = bts >= jnp.uint32(0x80000000)
      ukeys[pl.ds(j * L, L)] = jnp.where(neg, ~bts, bts | jnp.uint32(0x80000000))
      return 0

    with jax.named_scope("tk_keys"):
      lax.fori_loop(0, NVT, _init, 0, unroll=4)

    def _hz(i, _):
      hist[pl.ds(i * L, L)] = z16
      return 0

    lax.fori_loop(0, 256 // L, _hz, 0)

    def _hb(j, _):
      kv = ukeys[pl.ds(j * L, L)]
      bins = (kv >> jnp.uint32(24)).astype(jnp.int32)
      plsc.addupdate_scatter(hist, [bins], ones16)
      return 0

    with jax.named_scope("tk_hist"):
      lax.fori_loop(0, NVT, _hb, 0, unroll=4)

    def _bs(i, carry):
      acc, bucket, gabove = carry
      r = (256 // L - 1) - i
      h = hist[pl.ds(r * L, L)]
      hrev = lax.rev(h, (0,))
      rc = plsc.cumsum(hrev)
      sge = acc + rc
      sgt = sge - hrev
      hitm = jnp.logical_and(sgt < kk, sge >= kk)
      bucket_vals = r * L + (L - 1) - iota
      bucket = jnp.max(jnp.where(hitm, bucket_vals,
                                 jnp.broadcast_to(bucket, (L,))))
      gabove = jnp.max(jnp.where(hitm, sgt, jnp.broadcast_to(gabove, (L,))))
      return (acc + jnp.max(rc), bucket, gabove)

    with jax.named_scope("tk_bscan"):
      _, bucket, gabove = lax.fori_loop(
          0, 256 // L, _bs, (jnp.int32(0), jnp.int32(0), jnp.int32(0)))

    bucket_u = bucket.astype(jnp.uint32)

    def _cc(j, cpos):
      kv = ukeys[pl.ds(j * L, L)]
      m = (kv >> jnp.uint32(24)) == bucket_u
      plsc.store_compressed(cand.at[pl.ds(cpos, L)], kv, mask=m)
      return cpos + jnp.max(plsc.all_reduce_population_count(m))

    with jax.named_scope("tk_compact"):
      cpos = lax.fori_loop(0, NVT, _cc, jnp.int32(0), unroll=4)
    cand[pl.ds(cpos, L)] = jnp.zeros((L,), jnp.uint32)
    ncv = (cpos + (L - 1)) // L

    krem = kk - gabove
    lo0 = bucket_u << jnp.uint32(24)
    hi0 = lo0 + jnp.uint32(0x00FFFFFF)

    def bb(i, carry):
      lo, hi = carry
      cont = lo < hi
      mid = lo + ((hi - lo) >> jnp.uint32(1))

      def cb(j, acc):
        kv = cand[pl.ds(j * L, L)]
        return acc + plsc.all_reduce_population_count(kv > mid)

      cg = jnp.max(lax.fori_loop(0, ncv, cb, z16))
      smaller = cg < krem
      nlo = jnp.where(smaller, lo, mid + jnp.uint32(1))
      nhi = jnp.where(smaller, mid, hi)
      return (jnp.where(cont, nlo, lo), jnp.where(cont, nhi, hi))

    with jax.named_scope("tk_bisect"):
      _, thr = lax.fori_loop(0, 24, bb, (lo0, hi0))

    def cb2(j, acc):
      kv = ukeys[pl.ds(j * L, L)]
      return acc + plsc.all_reduce_population_count(kv > thr)

    with jax.named_scope("tk_counts"):
      agv = lax.fori_loop(0, NVT, cb2, z16, unroll=4)
    Gg = jnp.max(agv)
    need = kk - Gg

    WPT = K // SLOTS
    win_lo = slot * WPT
    win_hi = win_lo + WPT
    base_b = batch * T

    def comp(j, carry):
      rank, eqc = carry
      kv = ukeys[pl.ds(j * L, L)]
      gtm = kv > thr
      eqm = kv == thr
      gcnt = jnp.max(plsc.all_reduce_population_count(gtm))
      ecnt = jnp.max(plsc.all_reduce_population_count(eqm))
      etake = jnp.clip(need - eqc, 0, ecnt)
      tcnt = gcnt + etake

      @pl.when(jnp.logical_and(rank < win_hi, rank + tcnt > win_lo))
      def _():
        eq_i = eqm.astype(jnp.int32)
        incl_e = plsc.cumsum(eq_i)
        excl_e = incl_e - eq_i
        take_eq = jnp.logical_and(eqm, (eqc + excl_e) < need)
        takem = jnp.logical_or(gtm, take_eq)
        t_i = takem.astype(jnp.int32)
        incl_t = plsc.cumsum(t_i)
        excl_t = incl_t - t_i
        grank = rank + excl_t
        inwin = jnp.logical_and(
            takem,
            jnp.logical_and(grank >= win_lo, grank < win_hi))
        wpos = jnp.clip(rank - win_lo, 0, WPT)
        ridv = base_b + j * L + iota
        plsc.store_compressed(myrid.at[pl.ds(wpos, L)], ridv, mask=inwin)
        plsc.store_compressed(myval.at[pl.ds(wpos, L)],
                              svals[pl.ds(j * L, L)], mask=inwin)

      return (rank + tcnt, eqc + ecnt)

    with jax.named_scope("tk_comp"):
      lax.fori_loop(0, NVT, comp, (jnp.int32(0), jnp.int32(0)), unroll=2)

    with jax.named_scope("tk_scat"):
      dst0 = batch * KP + slot * WPT
      pltpu.sync_copy(myrid.at[pl.ds(0, WPT)], rid_hbm.at[pl.ds(dst0, WPT)])
      pltpu.sync_copy(myval.at[pl.ds(0, WPT)], val_hbm.at[pl.ds(dst0, WPT)])

    with jax.named_scope("tk_gather"):
      pltpu.async_copy(x_hbm.at[myrid.at[pl.ds(0, WPT)]], rows_v, sem).wait()
      pltpu.sync_copy(
          rows_v, sel_hbm.at[pl.ds((batch * SLOTS + slot) * WPT, WPT)])

  return _topk


def _make_gather(B, D, K, KP):
  RPT = (B * K) // (NC * NS)

  @functools.partial(
      pl.kernel,
      out_type=jax.ShapeDtypeStruct((B * K, D), jnp.float32),
      mesh=_sc_mesh(),
      scratch_types=[
          pltpu.VMEM((RPT,), jnp.int32),
          pltpu.VMEM((RPT, D), jnp.float32),
          pltpu.SemaphoreType.DMA,
      ],
  )
  def _gather(rid_hbm, x_hbm, sel_hbm, idx_v, rows_v, sem):
    c = lax.axis_index("c")
    s = lax.axis_index("s")
    t = c * NS + s
    batch = t // SLOTS
    seg = t % SLOTS
    pltpu.sync_copy(rid_hbm.at[pl.ds(batch * KP + seg * RPT, RPT)], idx_v)
    pltpu.async_copy(x_hbm.at[idx_v], rows_v, sem).wait()
    pltpu.sync_copy(rows_v, sel_hbm.at[pl.ds(t * RPT, RPT)])

  return _gather


def _make_scatter(B, D, K, KP):
  RPT = (B * K) // (NC * NS)

  @functools.partial(
      pl.kernel,
      out_type=(),
      mesh=_sc_mesh(),
      scratch_types=[
          pltpu.VMEM((RPT,), jnp.int32),
          pltpu.VMEM((RPT, D), jnp.float32),
          pltpu.SemaphoreType.DMA,
      ],
  )
  def _scatter(new_hbm, rid_hbm, out_hbm, idx_v, rows_v, sem):
    c = lax.axis_index("c")
    s = lax.axis_index("s")
    t = c * NS + s
    batch = t // SLOTS
    seg = t % SLOTS
    pltpu.sync_copy(rid_hbm.at[pl.ds(batch * KP + seg * RPT, RPT)], idx_v)
    pltpu.sync_copy(new_hbm.at[pl.ds(t * RPT, RPT)], rows_v)
    pltpu.async_copy(rows_v, out_hbm.at[idx_v], sem).wait()

  return _scatter


def kernel(x, w_router, W_layer, b_layer):
  B, T, D = x.shape
  K = max(1, int(T * 0.125))
  KP = K + 128
  R = B * T

  RB = 2048
  x2d_in = x.reshape(R, D)
  out2, scores2 = pl.pallas_call(
      _stage1_scores_copy,
      out_shape=[
          jax.ShapeDtypeStruct((R, D), jnp.float32),
          jax.ShapeDtypeStruct((R, 1), jnp.float32),
      ],
      grid=(R // RB,),
      in_specs=[
          pl.BlockSpec((D, 1), lambda i: (0, 0)),
          pl.BlockSpec((RB, D), lambda i: (i, 0)),
      ],
      out_specs=[
          pl.BlockSpec((RB, D), lambda i: (i, 0)),
          pl.BlockSpec((RB, 1), lambda i: (i, 0)),
      ],
  )(w_router.reshape(D, 1), x2d_in)
  scores_flat = scores2.reshape(R)
  out3 = out2

  x2d = x.reshape(R, D)
  rid_pad, val_pad, sel = _make_topk(B, T, D, K, KP)(scores_flat, x2d)

  vals2 = val_pad.reshape(B, KP)[:, :K].reshape(B * K, 1)
  RBM = 512
  newrows = pl.pallas_call(
      _stage4_layer,
      out_shape=jax.ShapeDtypeStruct((B * K, D), jnp.float32),
      grid=((B * K) // RBM,),
      in_specs=[
          pl.BlockSpec((RBM, D), lambda i: (i, 0)),
          pl.BlockSpec((D, D), lambda i: (0, 0)),
          pl.BlockSpec((1, D), lambda i: (0, 0)),
          pl.BlockSpec((RBM, 1), lambda i: (i, 0)),
      ],
      out_specs=pl.BlockSpec((RBM, D), lambda i: (i, 0)),
  )(sel, W_layer, b_layer.reshape(1, D), vals2)

  out_ref = jax.new_ref(out3.reshape(R, D))
  _make_scatter(B, D, K, KP)(newrows, rid_pad, out_ref)

  return jax.freeze(out_ref).reshape(B, T, D)


# speedup vs baseline: 1.0335x; 1.0335x over previous
import functools

import jax
import jax.numpy as jnp
from jax import lax
from jax.experimental import pallas as pl
from jax.experimental.pallas import tpu as pltpu
from jax.experimental.pallas import tpu_sc as plsc

L = 16
NC = 2
NS = 16
SLOTS = 8


def _sc_mesh():
  return plsc.VectorSubcoreMesh(
      core_axis_name="c", subcore_axis_name="s", num_cores=NC, num_subcores=NS)


def _stage1_scores_copy(w_ref, x_ref, o_ref, s_ref):
  xb = x_ref[...]
  o_ref[...] = xb
  s_ref[...] = jnp.dot(xb, w_ref[...], preferred_element_type=jnp.float32)


def _stage4_layer(sel_ref, w_ref, b_ref, v_ref, o_ref):
  sb = sel_ref[...]
  p = jnp.dot(sb, w_ref[...], preferred_element_type=jnp.float32) + b_ref[...]
  sg = jax.nn.sigmoid(v_ref[...])
  o_ref[...] = sb + (p * sg - sb * sg)


def _make_topk(B, T, D, K, KP):
  CH = T // SLOTS
  NV = CH // L
  NVT = T // L
  WPT = K // SLOTS

  @functools.partial(
      pl.kernel,
      out_type=[
          jax.ShapeDtypeStruct((B * KP,), jnp.int32),
          jax.ShapeDtypeStruct((B * KP,), jnp.float32),
          jax.ShapeDtypeStruct((B * K, D), jnp.float32),
      ],
      mesh=_sc_mesh(),
      compiler_params=pltpu.CompilerParams(needs_layout_passes=False),
      scratch_types=[
          pltpu.VMEM((T,), jnp.float32),
          pltpu.VMEM((T,), jnp.uint32),
          pltpu.VMEM((T + 4 * L,), jnp.uint32),
          pltpu.VMEM((256,), jnp.int32),
          pltpu.VMEM((WPT + L,), jnp.int32),
          pltpu.VMEM((WPT + L,), jnp.float32),
          pltpu.VMEM((WPT, D), jnp.float32),
          pltpu.SemaphoreType.DMA,
      ],
  )
  def _topk(scores_hbm, x_hbm, rid_hbm, val_hbm, sel_hbm, svals, ukeys, cand,
            hist, myrid, myval, rows_v, sem):
    c = lax.axis_index("c")
    s = lax.axis_index("s")
    bl = s // SLOTS
    slot = s % SLOTS
    batch = c * 2 + bl
    base_flat = batch * T + slot * CH

    with jax.named_scope("tk_load"):
      pltpu.sync_copy(scores_hbm.at[pl.ds(batch * T, T)], svals)

    iota = lax.iota(jnp.int32, L)
    z16 = jnp.zeros((L,), jnp.int32)
    ones16 = jnp.ones((L,), jnp.int32)
    kk = jnp.int32(K)

    def _init(j, _):
      v = svals[pl.ds(j * L, L)]
      bts = plsc.bitcast(v, jnp.uint32)
      neg = bts >= jnp.uint32(0x80000000)
      ukeys[pl.ds(j * L, L)] = jnp.where(neg, ~bts, bts | jnp.uint32(0x80000000))
      return 0

    with jax.named_scope("tk_keys"):
      lax.fori_loop(0, NVT, _init, 0, unroll=4)

    def _hz(i, _):
      hist[pl.ds(i * L, L)] = z16
      return 0

    lax.fori_loop(0, 256 // L, _hz, 0)

    def _hb(j, _):
      kv = ukeys[pl.ds(j * L, L)]
      bins = (kv >> jnp.uint32(24)).astype(jnp.int32)
      plsc.addupdate_scatter(hist, [bins], ones16)
      return 0

    with jax.named_scope("tk_hist"):
      lax.fori_loop(0, NVT, _hb, 0, unroll=4)

    def _bs(i, carry):
      acc, bucket, gabove = carry
      r = (256 // L - 1) - i
      h = hist[pl.ds(r * L, L)]
      hrev = lax.rev(h, (0,))
      rc = plsc.cumsum(hrev)
      sge = acc + rc
      sgt = sge - hrev
      hitm = jnp.logical_and(sgt < kk, sge >= kk)
      bucket_vals = r * L + (L - 1) - iota
      bucket = jnp.max(jnp.where(hitm, bucket_vals,
                                 jnp.broadcast_to(bucket, (L,))))
      gabove = jnp.max(jnp.where(hitm, sgt, jnp.broadcast_to(gabove, (L,))))
      return (acc + jnp.max(rc), bucket, gabove)

    with jax.named_scope("tk_bscan"):
      _, bucket, gabove = lax.fori_loop(
          0, 256 // L, _bs, (jnp.int32(0), jnp.int32(0), jnp.int32(0)))

    bucket_u = bucket.astype(jnp.uint32)

    def _cc(j, cpos):
      kv = ukeys[pl.ds(j * L, L)]
      m = (kv >> jnp.uint32(24)) == bucket_u
      plsc.store_compressed(cand.at[pl.ds(cpos, L)], kv, mask=m)
      return cpos + jnp.max(plsc.all_reduce_population_count(m))

    with jax.named_scope("tk_compact"):
      cpos = lax.fori_loop(0, NVT, _cc, jnp.int32(0), unroll=4)
    zpad = jnp.zeros((L,), jnp.uint32)
    cand[pl.ds(cpos, L)] = zpad
    cand[pl.ds(cpos + L, L)] = zpad
    cand[pl.ds(cpos + 2 * L, L)] = zpad
    cand[pl.ds(cpos + 3 * L, L)] = zpad
    ncv = (cpos + (L - 1)) // L

    krem = kk - gabove
    lo0 = bucket_u << jnp.uint32(24)
    hi0 = lo0 + jnp.uint32(0x00FFFFFF)

    def _mk_bb(count_fn):
      def bb(i, carry):
        lo, hi = carry
        cont = lo < hi
        mid = lo + ((hi - lo) >> jnp.uint32(1))
        cg = count_fn(mid)
        smaller = cg < krem
        nlo = jnp.where(smaller, lo, mid + jnp.uint32(1))
        nhi = jnp.where(smaller, mid, hi)
        return (jnp.where(cont, nlo, lo), jnp.where(cont, nhi, hi))
      return bb

    def _bisect_fast(_):
      def count4(mid):
        acc = z16
        for j in range(4):
          kv = cand[pl.ds(j * L, L)]
          acc = acc + plsc.all_reduce_population_count(kv > mid)
        return jnp.max(acc)
      _, t = lax.fori_loop(0, 24, _mk_bb(count4), (lo0, hi0))
      return t

    def _bisect_slow(_):
      def countn(mid):
        def cb(j, acc):
          kv = cand[pl.ds(j * L, L)]
          return acc + plsc.all_reduce_population_count(kv > mid)
        return jnp.max(lax.fori_loop(0, ncv, cb, z16))
      _, t = lax.fori_loop(0, 24, _mk_bb(countn), (lo0, hi0))
      return t

    with jax.named_scope("tk_bisect"):
      thr = lax.cond(cpos <= 64, _bisect_fast, _bisect_slow, 0)

    def cb2(j, acc):
      kv = ukeys[pl.ds(j * L, L)]
      return acc + plsc.all_reduce_population_count(kv > thr)

    with jax.named_scope("tk_counts"):
      agv = lax.fori_loop(0, NVT, cb2, z16, unroll=4)
    Gg = jnp.max(agv)
    need = kk - Gg

    WPT = K // SLOTS
    win_lo = slot * WPT
    win_hi = win_lo + WPT
    base_b = batch * T

    def comp(j, carry):
      rank, eqc, wpos = carry
      kv = ukeys[pl.ds(j * L, L)]
      gtm = kv > thr
      eqm = kv == thr
      eq_i = eqm.astype(jnp.int32)
      incl_e = plsc.cumsum(eq_i)
      excl_e = incl_e - eq_i
      take_eq = jnp.logical_and(eqm, (eqc + excl_e) < need)
      takem = jnp.logical_or(gtm, take_eq)
      t_i = takem.astype(jnp.int32)
      incl_t = plsc.cumsum(t_i)
      excl_t = incl_t - t_i
      grank = rank + excl_t
      inwin = jnp.logical_and(
          takem,
          jnp.logical_and(grank >= win_lo, grank < win_hi))
      ridv = base_b + j * L + iota
      plsc.store_compressed(myrid.at[pl.ds(wpos, L)], ridv, mask=inwin)
      plsc.store_compressed(myval.at[pl.ds(wpos, L)],
                            svals[pl.ds(j * L, L)], mask=inwin)
      nw = jnp.max(plsc.all_reduce_population_count(inwin))
      return (rank + jnp.max(incl_t), eqc + jnp.max(incl_e), wpos + nw)

    with jax.named_scope("tk_comp"):
      lax.fori_loop(0, NVT, comp,
                    (jnp.int32(0), jnp.int32(0), jnp.int32(0)), unroll=2)

    with jax.named_scope("tk_scat"):
      dst0 = batch * KP + slot * WPT
      pltpu.sync_copy(myrid.at[pl.ds(0, WPT)], rid_hbm.at[pl.ds(dst0, WPT)])
      pltpu.sync_copy(myval.at[pl.ds(0, WPT)], val_hbm.at[pl.ds(dst0, WPT)])

    with jax.named_scope("tk_gather"):
      pltpu.async_copy(x_hbm.at[myrid.at[pl.ds(0, WPT)]], rows_v, sem).wait()
      pltpu.sync_copy(
          rows_v, sel_hbm.at[pl.ds((batch * SLOTS + slot) * WPT, WPT)])

  return _topk


def _make_gather(B, D, K, KP):
  RPT = (B * K) // (NC * NS)

  @functools.partial(
      pl.kernel,
      out_type=jax.ShapeDtypeStruct((B * K, D), jnp.float32),
      mesh=_sc_mesh(),
      scratch_types=[
          pltpu.VMEM((RPT,), jnp.int32),
          pltpu.VMEM((RPT, D), jnp.float32),
          pltpu.SemaphoreType.DMA,
      ],
  )
  def _gather(rid_hbm, x_hbm, sel_hbm, idx_v, rows_v, sem):
    c = lax.axis_index("c")
    s = lax.axis_index("s")
    t = c * NS + s
    batch = t // SLOTS
    seg = t % SLOTS
    pltpu.sync_copy(rid_hbm.at[pl.ds(batch * KP + seg * RPT, RPT)], idx_v)
    pltpu.async_copy(x_hbm.at[idx_v], rows_v, sem).wait()
    pltpu.sync_copy(rows_v, sel_hbm.at[pl.ds(t * RPT, RPT)])

  return _gather


def _make_scatter(B, D, K, KP):
  RPT = (B * K) // (NC * NS)

  @functools.partial(
      pl.kernel,
      out_type=(),
      mesh=_sc_mesh(),
      scratch_types=[
          pltpu.VMEM((RPT,), jnp.int32),
          pltpu.VMEM((RPT, D), jnp.float32),
          pltpu.SemaphoreType.DMA,
      ],
  )
  def _scatter(new_hbm, rid_hbm, out_hbm, idx_v, rows_v, sem):
    c = lax.axis_index("c")
    s = lax.axis_index("s")
    t = c * NS + s
    batch = t // SLOTS
    seg = t % SLOTS
    pltpu.sync_copy(rid_hbm.at[pl.ds(batch * KP + seg * RPT, RPT)], idx_v)
    pltpu.sync_copy(new_hbm.at[pl.ds(t * RPT, RPT)], rows_v)
    pltpu.async_copy(rows_v, out_hbm.at[idx_v], sem).wait()

  return _scatter


def kernel(x, w_router, W_layer, b_layer):
  B, T, D = x.shape
  K = max(1, int(T * 0.125))
  KP = K + 128
  R = B * T

  RB = 1024
  x2d_in = x.reshape(R, D)
  out2, scores2 = pl.pallas_call(
      _stage1_scores_copy,
      out_shape=[
          jax.ShapeDtypeStruct((R, D), jnp.float32),
          jax.ShapeDtypeStruct((R, 1), jnp.float32),
      ],
      grid=(R // RB,),
      in_specs=[
          pl.BlockSpec((D, 1), lambda i: (0, 0)),
          pl.BlockSpec((RB, D), lambda i: (i, 0)),
      ],
      out_specs=[
          pl.BlockSpec((RB, D), lambda i: (i, 0)),
          pl.BlockSpec((RB, 1), lambda i: (i, 0)),
      ],
  )(w_router.reshape(D, 1), x2d_in)
  scores_flat = scores2.reshape(R)
  out3 = out2

  x2d = x.reshape(R, D)
  rid_pad, val_pad, sel = _make_topk(B, T, D, K, KP)(scores_flat, x2d)

  vals2 = val_pad.reshape(B, KP)[:, :K].reshape(B * K, 1)
  RBM = 512
  newrows = pl.pallas_call(
      _stage4_layer,
      out_shape=jax.ShapeDtypeStruct((B * K, D), jnp.float32),
      grid=((B * K) // RBM,),
      in_specs=[
          pl.BlockSpec((RBM, D), lambda i: (i, 0)),
          pl.BlockSpec((D, D), lambda i: (0, 0)),
          pl.BlockSpec((1, D), lambda i: (0, 0)),
          pl.BlockSpec((RBM, 1), lambda i: (i, 0)),
      ],
      out_specs=pl.BlockSpec((RBM, D), lambda i: (i, 0)),
  )(sel, W_layer, b_layer.reshape(1, D), vals2)

  out_ref = jax.new_ref(out3.reshape(R, D))
  _make_scatter(B, D, K, KP)(newrows, rid_pad, out_ref)

  return jax.freeze(out_ref).reshape(B, T, D)


# speedup vs baseline: 1.0347x; 1.0012x over previous
import functools

import jax
import jax.numpy as jnp
from jax import lax
from jax.experimental import pallas as pl
from jax.experimental.pallas import tpu as pltpu
from jax.experimental.pallas import tpu_sc as plsc

L = 16
NC = 2
NS = 16
SLOTS = 8


def _sc_mesh():
  return plsc.VectorSubcoreMesh(
      core_axis_name="c", subcore_axis_name="s", num_cores=NC, num_subcores=NS)


def _stage1_scores_copy(w_ref, x_ref, o_ref, s_ref):
  xb = x_ref[...]
  o_ref[...] = xb
  s_ref[...] = jnp.dot(xb, w_ref[...], preferred_element_type=jnp.float32)


def _stage4_layer(sel_ref, w_ref, b_ref, v_ref, o_ref):
  sb = sel_ref[...]
  p = jnp.dot(sb, w_ref[...], preferred_element_type=jnp.float32) + b_ref[...]
  sg = jax.nn.sigmoid(v_ref[...])
  o_ref[...] = sb + (p * sg - sb * sg)


def _make_topk(B, T, D, K, KP):
  CH = T // SLOTS
  NV = CH // L
  NVT = T // L
  WPT = K // SLOTS

  @functools.partial(
      pl.kernel,
      out_type=[
          jax.ShapeDtypeStruct((B * KP,), jnp.int32),
          jax.ShapeDtypeStruct((B * KP,), jnp.float32),
          jax.ShapeDtypeStruct((B * K, D), jnp.float32),
      ],
      mesh=_sc_mesh(),
      compiler_params=pltpu.CompilerParams(needs_layout_passes=False),
      scratch_types=[
          pltpu.VMEM((T,), jnp.float32),
          pltpu.VMEM((T,), jnp.uint32),
          pltpu.VMEM((T + 4 * L,), jnp.uint32),
          pltpu.VMEM((256,), jnp.int32),
          pltpu.VMEM((WPT + L,), jnp.int32),
          pltpu.VMEM((WPT + L,), jnp.float32),
          pltpu.VMEM((WPT, D), jnp.float32),
          pltpu.SemaphoreType.DMA,
      ],
  )
  def _topk(scores_hbm, x_hbm, rid_hbm, val_hbm, sel_hbm, svals, ukeys, cand,
            hist, myrid, myval, rows_v, sem):
    c = lax.axis_index("c")
    s = lax.axis_index("s")
    bl = s // SLOTS
    slot = s % SLOTS
    batch = c * 2 + bl
    base_flat = batch * T + slot * CH

    with jax.named_scope("tk_load"):
      pltpu.sync_copy(scores_hbm.at[pl.ds(batch * T, T)], svals)

    iota = lax.iota(jnp.int32, L)
    z16 = jnp.zeros((L,), jnp.int32)
    ones16 = jnp.ones((L,), jnp.int32)
    kk = jnp.int32(K)

    def _init(j, _):
      v = svals[pl.ds(j * L, L)]
      bts = plsc.bitcast(v, jnp.uint32)
      neg = bts >= jnp.uint32(0x80000000)
      ukeys[pl.ds(j * L, L)] = jnp.where(neg, ~bts, bts | jnp.uint32(0x80000000))
      return 0

    with jax.named_scope("tk_keys"):
      lax.fori_loop(0, NVT, _init, 0, unroll=4)

    def _hz(i, _):
      hist[pl.ds(i * L, L)] = z16
      return 0

    lax.fori_loop(0, 256 // L, _hz, 0)

    def _hb(j, _):
      kv = ukeys[pl.ds(j * L, L)]
      bins = (kv >> jnp.uint32(24)).astype(jnp.int32)
      plsc.addupdate_scatter(hist, [bins], ones16)
      return 0

    with jax.named_scope("tk_hist"):
      lax.fori_loop(0, NVT, _hb, 0, unroll=4)

    def _bs(i, carry):
      acc, bucket, gabove = carry
      r = (256 // L - 1) - i
      h = hist[pl.ds(r * L, L)]
      hrev = lax.rev(h, (0,))
      rc = plsc.cumsum(hrev)
      sge = acc + rc
      sgt = sge - hrev
      hitm = jnp.logical_and(sgt < kk, sge >= kk)
      bucket_vals = r * L + (L - 1) - iota
      bucket = jnp.max(jnp.where(hitm, bucket_vals,
                                 jnp.broadcast_to(bucket, (L,))))
      gabove = jnp.max(jnp.where(hitm, sgt, jnp.broadcast_to(gabove, (L,))))
      return (acc + jnp.max(rc), bucket, gabove)

    with jax.named_scope("tk_bscan"):
      _, bucket, gabove = lax.fori_loop(
          0, 256 // L, _bs, (jnp.int32(0), jnp.int32(0), jnp.int32(0)))

    bucket_u = bucket.astype(jnp.uint32)

    def _cc(j, cpos):
      kv = ukeys[pl.ds(j * L, L)]
      m = (kv >> jnp.uint32(24)) == bucket_u
      plsc.store_compressed(cand.at[pl.ds(cpos, L)], kv, mask=m)
      return cpos + jnp.max(plsc.all_reduce_population_count(m))

    with jax.named_scope("tk_compact"):
      cpos = lax.fori_loop(0, NVT, _cc, jnp.int32(0), unroll=4)
    zpad = jnp.zeros((L,), jnp.uint32)
    cand[pl.ds(cpos, L)] = zpad
    cand[pl.ds(cpos + L, L)] = zpad
    cand[pl.ds(cpos + 2 * L, L)] = zpad
    cand[pl.ds(cpos + 3 * L, L)] = zpad
    ncv = (cpos + (L - 1)) // L

    krem = kk - gabove
    lo0 = bucket_u << jnp.uint32(24)
    hi0 = lo0 + jnp.uint32(0x00FFFFFF)

    krem_v = jnp.broadcast_to(krem, (L,))
    lo0_v = jnp.broadcast_to(lo0, (L,))
    hi0_v = jnp.broadcast_to(hi0, (L,))
    one_u = jnp.broadcast_to(jnp.uint32(1), (L,))

    def _mk_bb(count_fn):
      def bb(i, carry):
        lo, hi = carry
        cont = lo < hi
        mid = lo + ((hi - lo) >> one_u)
        cg = count_fn(mid)
        smaller = cg < krem_v
        nlo = jnp.where(smaller, lo, mid + one_u)
        nhi = jnp.where(smaller, mid, hi)
        return (jnp.where(cont, nlo, lo), jnp.where(cont, nhi, hi))
      return bb

    def _bisect_fast(_):
      def count4(mid):
        acc = z16
        for j in range(4):
          kv = cand[pl.ds(j * L, L)]
          acc = acc + plsc.all_reduce_population_count(kv > mid)
        return acc
      _, t = lax.fori_loop(0, 24, _mk_bb(count4), (lo0_v, hi0_v))
      return t

    def _bisect_slow(_):
      def countn(mid):
        def cb(j, acc):
          kv = cand[pl.ds(j * L, L)]
          return acc + plsc.all_reduce_population_count(kv > mid)
        return lax.fori_loop(0, ncv, cb, z16)
      _, t = lax.fori_loop(0, 24, _mk_bb(countn), (lo0_v, hi0_v))
      return t

    with jax.named_scope("tk_bisect"):
      thr = lax.cond(cpos <= 64, _bisect_fast, _bisect_slow, 0)

    def cb2(j, acc):
      kv = ukeys[pl.ds(j * L, L)]
      return acc + plsc.all_reduce_population_count(kv > thr)

    with jax.named_scope("tk_counts"):
      agv = lax.fori_loop(0, NVT, cb2, z16, unroll=4)
    kk_v = jnp.broadcast_to(kk, (L,))
    need_v = kk_v - agv

    WPT = K // SLOTS
    win_lo = slot * WPT
    win_hi = win_lo + WPT
    base_b = batch * T

    win_lo_v = jnp.broadcast_to(win_lo, (L,))
    win_hi_v = jnp.broadcast_to(win_hi, (L,))

    def comp(j, carry):
      rank_v, eqc_v = carry
      kv = ukeys[pl.ds(j * L, L)]
      gtm = kv > thr
      eqm = kv == thr
      eq_i = eqm.astype(jnp.int32)
      incl_e = plsc.cumsum(eq_i)
      excl_e = incl_e - eq_i
      take_eq = jnp.logical_and(eqm, (eqc_v + excl_e) < need_v)
      takem = jnp.logical_or(gtm, take_eq)
      t_i = takem.astype(jnp.int32)
      incl_t = plsc.cumsum(t_i)
      excl_t = incl_t - t_i
      grank = rank_v + excl_t
      inwin = jnp.logical_and(
          takem,
          jnp.logical_and(grank >= win_lo_v, grank < win_hi_v))
      dst = jnp.clip(grank - win_lo_v, 0, WPT - 1)
      ridv = base_b + j * L + iota
      plsc.store_scatter(myrid, [dst], ridv, mask=inwin)
      plsc.store_scatter(myval, [dst], svals[pl.ds(j * L, L)], mask=inwin)
      return (rank_v + plsc.all_reduce_population_count(takem),
              eqc_v + plsc.all_reduce_population_count(eqm))

    with jax.named_scope("tk_comp"):
      lax.fori_loop(0, NVT, comp, (z16, z16), unroll=2)

    with jax.named_scope("tk_scat"):
      dst0 = batch * KP + slot * WPT
      pltpu.sync_copy(myrid.at[pl.ds(0, WPT)], rid_hbm.at[pl.ds(dst0, WPT)])
      pltpu.sync_copy(myval.at[pl.ds(0, WPT)], val_hbm.at[pl.ds(dst0, WPT)])

    with jax.named_scope("tk_gather"):
      pltpu.async_copy(x_hbm.at[myrid.at[pl.ds(0, WPT)]], rows_v, sem).wait()
      pltpu.sync_copy(
          rows_v, sel_hbm.at[pl.ds((batch * SLOTS + slot) * WPT, WPT)])

  return _topk


def _make_gather(B, D, K, KP):
  RPT = (B * K) // (NC * NS)

  @functools.partial(
      pl.kernel,
      out_type=jax.ShapeDtypeStruct((B * K, D), jnp.float32),
      mesh=_sc_mesh(),
      scratch_types=[
          pltpu.VMEM((RPT,), jnp.int32),
          pltpu.VMEM((RPT, D), jnp.float32),
          pltpu.SemaphoreType.DMA,
      ],
  )
  def _gather(rid_hbm, x_hbm, sel_hbm, idx_v, rows_v, sem):
    c = lax.axis_index("c")
    s = lax.axis_index("s")
    t = c * NS + s
    batch = t // SLOTS
    seg = t % SLOTS
    pltpu.sync_copy(rid_hbm.at[pl.ds(batch * KP + seg * RPT, RPT)], idx_v)
    pltpu.async_copy(x_hbm.at[idx_v], rows_v, sem).wait()
    pltpu.sync_copy(rows_v, sel_hbm.at[pl.ds(t * RPT, RPT)])

  return _gather


def _make_scatter(B, D, K, KP):
  RPT = (B * K) // (NC * NS)

  @functools.partial(
      pl.kernel,
      out_type=(),
      mesh=_sc_mesh(),
      scratch_types=[
          pltpu.VMEM((RPT,), jnp.int32),
          pltpu.VMEM((RPT, D), jnp.float32),
          pltpu.SemaphoreType.DMA,
      ],
  )
  def _scatter(new_hbm, rid_hbm, out_hbm, idx_v, rows_v, sem):
    c = lax.axis_index("c")
    s = lax.axis_index("s")
    t = c * NS + s
    batch = t // SLOTS
    seg = t % SLOTS
    pltpu.sync_copy(rid_hbm.at[pl.ds(batch * KP + seg * RPT, RPT)], idx_v)
    pltpu.sync_copy(new_hbm.at[pl.ds(t * RPT, RPT)], rows_v)
    pltpu.async_copy(rows_v, out_hbm.at[idx_v], sem).wait()

  return _scatter


def kernel(x, w_router, W_layer, b_layer):
  B, T, D = x.shape
  K = max(1, int(T * 0.125))
  KP = K + 128
  R = B * T

  RB = 1024
  x2d_in = x.reshape(R, D)
  out2, scores2 = pl.pallas_call(
      _stage1_scores_copy,
      out_shape=[
          jax.ShapeDtypeStruct((R, D), jnp.float32),
          jax.ShapeDtypeStruct((R, 1), jnp.float32),
      ],
      grid=(R // RB,),
      in_specs=[
          pl.BlockSpec((D, 1), lambda i: (0, 0)),
          pl.BlockSpec((RB, D), lambda i: (i, 0)),
      ],
      out_specs=[
          pl.BlockSpec((RB, D), lambda i: (i, 0)),
          pl.BlockSpec((RB, 1), lambda i: (i, 0)),
      ],
  )(w_router.reshape(D, 1), x2d_in)
  scores_flat = scores2.reshape(R)
  out3 = out2

  x2d = x.reshape(R, D)
  rid_pad, val_pad, sel = _make_topk(B, T, D, K, KP)(scores_flat, x2d)

  vals2 = val_pad.reshape(B, KP)[:, :K].reshape(B * K, 1)
  RBM = 512
  newrows = pl.pallas_call(
      _stage4_layer,
      out_shape=jax.ShapeDtypeStruct((B * K, D), jnp.float32),
      grid=((B * K) // RBM,),
      in_specs=[
          pl.BlockSpec((RBM, D), lambda i: (i, 0)),
          pl.BlockSpec((D, D), lambda i: (0, 0)),
          pl.BlockSpec((1, D), lambda i: (0, 0)),
          pl.BlockSpec((RBM, 1), lambda i: (i, 0)),
      ],
      out_specs=pl.BlockSpec((RBM, D), lambda i: (i, 0)),
  )(sel, W_layer, b_layer.reshape(1, D), vals2)

  out_ref = jax.new_ref(out3.reshape(R, D))
  _make_scatter(B, D, K, KP)(newrows, rid_pad, out_ref)

  return jax.freeze(out_ref).reshape(B, T, D)
